# baseline (speedup 1.0000x reference)
import functools

import jax
import jax.numpy as jnp
from jax import lax
from jax.experimental import pallas as pl
from jax.experimental.pallas import tpu as pltpu
from jax.experimental.pallas import tpu_sc as plsc

_EPS = 1e-5


def _mm_bias_pack_body(x_ref, w_ref, b_ref, o_ref):
    h = (jnp.dot(x_ref[:, :], w_ref[:, :], preferred_element_type=jnp.float32)
         + b_ref[:, :])
    half = h.shape[1] // 2
    ai = lax.bitcast_convert_type(h[:, :half], jnp.int32)
    bi = lax.bitcast_convert_type(h[:, half:], jnp.int32)
    ta = (ai + 0x7FFF + ((ai >> 16) & 1)) >> 16
    tb = (bi + 0x7FFF + ((bi >> 16) & 1)) >> 16
    word = (tb << 16) | (ta & 0xFFFF)
    o_ref[:, :] = lax.bitcast_convert_type(word, jnp.float32)


def _mm_bias_pack(x, W, b, block):
    n, k = x.shape
    m = W.shape[1]
    return pl.pallas_call(
        _mm_bias_pack_body,
        grid=(n // block,),
        in_specs=[
            pl.BlockSpec((block, k), lambda i: (i, 0)),
            pl.BlockSpec((k, m), lambda i: (0, 0)),
            pl.BlockSpec((1, m), lambda i: (0, 0)),
        ],
        out_specs=pl.BlockSpec((block, m // 2), lambda i: (i, 0)),
        out_shape=jax.ShapeDtypeStruct((n, m // 2), jnp.float32),
    )(x, W, b.reshape(1, m))


def _ln(r, g, be):
    m = jnp.mean(r, axis=1, keepdims=True)
    d = r - m
    v = jnp.mean(d * d, axis=1, keepdims=True)
    return d * lax.rsqrt(v + _EPS) * g + be


def _edge_body(h_ref, a_ref, wl_ref, g_ref, be_ref, o_ref):
    w = lax.bitcast_convert_type(h_ref[:, :], jnp.int32)
    lo = lax.bitcast_convert_type(w << 16, jnp.float32)
    hi = lax.bitcast_convert_type(w & jnp.int32(-65536), jnp.float32)
    h = jnp.concatenate([lo, hi], axis=1)
    contrib = lax.dot_general(a_ref[0], wl_ref[:, :],
                              (((0,), (0,)), ((), ())),
                              preferred_element_type=jnp.float32)
    z = h + contrib
    r = jnp.maximum(z, 0.0)
    o_ref[:, :] = _ln(r, g_ref[:, :], be_ref[:, :])


def _edge_elem(Hrow, attr_row, w1l, g1, be1, block):
    e, dp = Hrow.shape
    d = dp * 2
    attr3 = attr_row.reshape(e // block, 1, block)
    return pl.pallas_call(
        _edge_body,
        grid=(e // block,),
        in_specs=[
            pl.BlockSpec((block, dp), lambda i: (i, 0)),
            pl.BlockSpec((1, 1, block), lambda i: (i, 0, 0)),
            pl.BlockSpec((1, d), lambda i: (0, 0)),
            pl.BlockSpec((1, d), lambda i: (0, 0)),
            pl.BlockSpec((1, d), lambda i: (0, 0)),
        ],
        out_specs=pl.BlockSpec((block, d), lambda i: (i, 0)),
        out_shape=jax.ShapeDtypeStruct((e, d), jnp.float32),
    )(Hrow, attr3, w1l.reshape(1, d), g1.reshape(1, d), be1.reshape(1, d))


def _node_pre_body(x_ref, bt_ref, u_ref, wx_ref, wu_ref, b_ref, o_ref):
    gsize = u_ref.shape[1]
    onehot = jnp.where(
        bt_ref[:, :] == lax.broadcasted_iota(jnp.int32, (1, gsize), 1), 1.0, 0.0)
    ub = jnp.sum(onehot * u_ref[:, :], axis=1, keepdims=True)
    o_ref[:, :] = (
        jnp.dot(x_ref[:, :], wx_ref[:, :], preferred_element_type=jnp.float32)
        + ub * wu_ref[:, :] + b_ref[:, :])


def _node_pre(x, u, batch, W2, b2, block):
    n, d = x.shape
    g = u.shape[0]
    wx = W2[:d]
    wu = W2[2 * d:2 * d + 1].reshape(1, d)
    return pl.pallas_call(
        _node_pre_body,
        grid=(n // block,),
        in_specs=[
            pl.BlockSpec((block, d), lambda i: (i, 0)),
            pl.BlockSpec((block, 1), lambda i: (i, 0)),
            pl.BlockSpec((1, g), lambda i: (0, 0)),
            pl.BlockSpec((d, d), lambda i: (0, 0)),
            pl.BlockSpec((1, d), lambda i: (0, 0)),
            pl.BlockSpec((1, d), lambda i: (0, 0)),
        ],
        out_specs=pl.BlockSpec((block, d), lambda i: (i, 0)),
        out_shape=jax.ShapeDtypeStruct((n, d), jnp.float32),
    )(x, batch.reshape(n, 1), u.reshape(1, g), wx, wu, b2.reshape(1, d))


def _node_body(*refs):
    nparts = len(refs) - 6
    p_ref = refs[0]
    s_refs = refs[1:1 + nparts]
    c_ref, wm_ref, g_ref, be_ref, o_ref = refs[1 + nparts:]
    cnt = c_ref[0, :, :1] + c_ref[1, :, :1]
    s = s_refs[0][:, :].astype(jnp.float32)
    for sr in s_refs[1:]:
        s = s + sr[:, :].astype(jnp.float32)
    mean = s / jnp.maximum(cnt, 1.0)
    z = p_ref[:, :] + jnp.dot(mean, wm_ref[:, :],
                              preferred_element_type=jnp.float32)
    r = jnp.maximum(z, 0.0)
    o_ref[:, :] = _ln(r, g_ref[:, :], be_ref[:, :])


def _node_mlp(pre, s_list, cnt16, W2, g2, be2, block):
    n, d = pre.shape
    wm = W2[d:2 * d]
    return pl.pallas_call(
        _node_body,
        grid=(n // block,),
        in_specs=[
            pl.BlockSpec((block, d), lambda i: (i, 0)),
            *[pl.BlockSpec((block, d), lambda i: (i, 0)) for _ in s_list],
            pl.BlockSpec((2, block, 16), lambda i: (0, i, 0)),
            pl.BlockSpec((d, d), lambda i: (0, 0)),
            pl.BlockSpec((1, d), lambda i: (0, 0)),
            pl.BlockSpec((1, d), lambda i: (0, 0)),
        ],
        out_specs=pl.BlockSpec((block, d), lambda i: (i, 0)),
        out_shape=jax.ShapeDtypeStruct((n, d), jnp.float32),
    )(pre, *s_list, cnt16, wm, g2.reshape(1, d), be2.reshape(1, d))


_NW = 32
_GCHUNK = 200


def _sc_gather(H, row):
    e = row.shape[0]
    d = H.shape[1]
    per_w = e // _NW
    steps = per_w // _GCHUNK
    mesh = plsc.VectorSubcoreMesh(core_axis_name="c", subcore_axis_name="s")

    ch = _GCHUNK
    assert steps >= 4
    odd = steps % 2 == 1

    @functools.partial(
        pl.kernel, mesh=mesh,
        out_type=jax.ShapeDtypeStruct((e, d), jnp.float32),
        compiler_params=pltpu.CompilerParams(use_tc_tiling_on_sc=True),
        scratch_types=[
            pltpu.VMEM((ch,), jnp.int32),
            pltpu.VMEM((ch,), jnp.int32),
            pltpu.VMEM((ch, d), jnp.float32),
            pltpu.VMEM((ch, d), jnp.float32),
            pltpu.SemaphoreType.DMA,
            pltpu.SemaphoreType.DMA,
            pltpu.SemaphoreType.DMA,
            pltpu.SemaphoreType.DMA,
        ],
    )
    def k(h_hbm, row_hbm, out_hbm, ia, ib, ra, rb, sga, sgb, swa, swb):
        wid = lax.axis_index("s") * 2 + lax.axis_index("c")
        base = wid * per_w

        def off(kk):
            return pl.multiple_of(base + kk * ch, 8)

        def load_idx(kk, iv):
            pltpu.sync_copy(row_hbm.at[pl.ds(off(kk), ch)], iv)

        def start_g(iv, rv, sem):
            pltpu.async_copy(h_hbm.at[iv], rv, sem)

        def wait_g(iv, rv, sem):
            pltpu.make_async_copy(h_hbm.at[iv], rv, sem).wait()

        def start_w(kk, rv, sem):
            pltpu.async_copy(rv, out_hbm.at[pl.ds(off(kk), ch)], sem)

        def wait_w(kk, rv, sem):
            pltpu.make_async_copy(rv, out_hbm.at[pl.ds(off(kk), ch)], sem).wait()

        load_idx(0, ia)
        start_g(ia, ra, sga)
        load_idx(1, ib)
        start_g(ib, rb, sgb)

        def body(m, carry):
            k0 = 2 * m
            k1 = k0 + 1
            wait_g(ia, ra, sga)
            start_w(k0, ra, swa)
            wait_g(ib, rb, sgb)
            start_w(k1, rb, swb)
            load_idx(k0 + 2, ia)
            wait_w(k0, ra, swa)
            start_g(ia, ra, sga)
            load_idx(k1 + 2, ib)
            wait_w(k1, rb, swb)
            start_g(ib, rb, sgb)
            return carry

        if odd:
            lax.fori_loop(0, (steps - 3) // 2, body, 0)
            k2 = steps - 3
            wait_g(ia, ra, sga)
            start_w(k2, ra, swa)
            load_idx(steps - 1, ia)
            wait_w(k2, ra, swa)
            start_g(ia, ra, sga)
            wait_g(ib, rb, sgb)
            start_w(k2 + 1, rb, swb)
            wait_g(ia, ra, sga)
            start_w(k2 + 2, ra, swa)
            wait_w(k2 + 2, ra, swa)
            wait_w(k2 + 1, rb, swb)
        else:
            lax.fori_loop(0, (steps - 2) // 2, body, 0)
            wait_g(ia, ra, sga)
            start_w(steps - 2, ra, swa)
            wait_g(ib, rb, sgb)
            start_w(steps - 1, rb, swb)
            wait_w(steps - 2, ra, swa)
            wait_w(steps - 1, rb, swb)

    return k(H, row)


_SCHUNK = 200
_DCHUNK = 192


def _sc_scatter(Eout, col, n):
    e, d = Eout.shape
    half = d // 2
    per_tile = e // 16
    ch = next(c for c in range(_DCHUNK, 40, -8) if per_tile % c == 0)
    steps = per_tile // ch
    assert steps >= 4
    odd = steps % 2 == 1
    n_wr = n // 10
    q_init = n_wr // ch
    r_init = n_wr % ch
    assert r_init % 8 == 0
    mesh = plsc.VectorSubcoreMesh(core_axis_name="c", subcore_axis_name="s")

    @functools.partial(
        pl.kernel, mesh=mesh,
        out_type=jax.ShapeDtypeStruct((n, d), jnp.float32),
        compiler_params=pltpu.CompilerParams(use_tc_tiling_on_sc=True),
        scratch_types=[
            pltpu.VMEM((ch,), jnp.int32),
            pltpu.VMEM((ch,), jnp.int32),
            pltpu.VMEM((ch, half), jnp.float32),
            pltpu.VMEM((ch, half), jnp.float32),
            pltpu.VMEM_SHARED((n, half), jnp.float32),
            pltpu.SemaphoreType.DMA,
            pltpu.SemaphoreType.DMA,
            pltpu.SemaphoreType.DMA,
            pltpu.SemaphoreType.DMA,
        ],
    )
    def k(eo_hbm, col_hbm, s_hbm, ia, ib, da, db, acc_sh, sra, srb, saa, sab):
        tid = lax.axis_index("s")
        core = lax.axis_index("c")
        f0 = pl.multiple_of(core * half, half)
        wbase = pl.multiple_of(tid * n_wr, 8)

        zero16 = jnp.zeros((16,), jnp.float32)

        def zdat_body(r, carry):
            for c8 in range(half // 16):
                da[r, pl.ds(c8 * 16, 16)] = zero16
            return carry
        lax.fori_loop(0, ch, zdat_body, 0)

        @pl.when(tid < 10)
        def _():
            for j in range(q_init):
                pltpu.sync_copy(da, acc_sh.at[pl.ds(wbase + j * ch, ch)])
            if r_init:
                pltpu.sync_copy(
                    da.at[pl.ds(0, r_init)],
                    acc_sh.at[pl.ds(wbase + q_init * ch, r_init)])

        plsc.subcore_barrier()

        ebase = tid * per_tile

        def off(kk):
            return pl.multiple_of(ebase + kk * ch, 8)

        def fetch(kk, iv, dv, sem):
            pltpu.sync_copy(col_hbm.at[pl.ds(off(kk), ch)], iv)
            pltpu.async_copy(
                eo_hbm.at[pl.ds(off(kk), ch), pl.ds(f0, half)], dv, sem)

        def wait_r(kk, dv, sem):
            pltpu.make_async_copy(
                eo_hbm.at[pl.ds(off(kk), ch), pl.ds(f0, half)], dv,
                sem).wait()

        def start_a(iv, dv, sem):
            pltpu.async_copy(dv, acc_sh.at[iv], sem, add=True)

        def wait_a(iv, dv, sem):
            pltpu.make_async_copy(dv, acc_sh.at[iv], sem).wait()

        fetch(0, ia, da, sra)
        fetch(1, ib, db, srb)

        def body(m, carry):
            k0 = 2 * m
            k1 = k0 + 1
            wait_r(k0, da, sra)
            start_a(ia, da, saa)
            wait_r(k1, db, srb)
            start_a(ib, db, sab)
            wait_a(ia, da, saa)
            fetch(k0 + 2, ia, da, sra)
            wait_a(ib, db, sab)
            fetch(k1 + 2, ib, db, srb)
            return carry

        if odd:
            lax.fori_loop(0, (steps - 3) // 2, body, 0)
            k2 = steps - 3
            wait_r(k2, da, sra)
            start_a(ia, da, saa)
            wait_a(ia, da, saa)
            fetch(steps - 1, ia, da, sra)
            wait_r(k2 + 1, db, srb)
            start_a(ib, db, sab)
            wait_r(k2 + 2, da, sra)
            start_a(ia, da, saa)
            wait_a(ia, da, saa)
            wait_a(ib, db, sab)
        else:
            lax.fori_loop(0, (steps - 2) // 2, body, 0)
            wait_r(steps - 2, da, sra)
            start_a(ia, da, saa)
            wait_r(steps - 1, db, srb)
            start_a(ib, db, sab)
            wait_a(ia, da, saa)
            wait_a(ib, db, sab)

        plsc.subcore_barrier()

        @pl.when(tid < 10)
        def _():
            pltpu.sync_copy(acc_sh.at[pl.ds(wbase, n_wr)],
                            s_hbm.at[pl.ds(wbase, n_wr), pl.ds(f0, half)])

    return k(Eout, col)


def _sc_counts(col, n):
    e = col.shape[0]
    per_w = e // _NW
    cch = 1000
    steps = per_w // cch
    n_wr = n // 10
    mesh = plsc.VectorSubcoreMesh(core_axis_name="c", subcore_axis_name="s")

    @functools.partial(
        pl.kernel, mesh=mesh,
        out_type=jax.ShapeDtypeStruct((2, n, 16), jnp.float32),
        compiler_params=pltpu.CompilerParams(use_tc_tiling_on_sc=False),
        scratch_types=[
            pltpu.VMEM((cch,), jnp.int32),
            pltpu.VMEM((cch, 16), jnp.float32),
            pltpu.VMEM_SHARED((n, 16), jnp.float32),
        ],
    )
    def k(col_hbm, cnt_hbm, idx_v, ones_v, cnt_sh):
        tid = lax.axis_index("s")
        core = lax.axis_index("c")
        wid = tid * 2 + core
        wbase = pl.multiple_of(tid * n_wr, 8)

        zero16 = jnp.zeros((16,), jnp.float32)
        one16 = jnp.ones((16,), jnp.float32)
        n_init = n_wr // cch

        def zb_body(r, carry):
            ones_v[r] = zero16
            return carry
        lax.fori_loop(0, cch, zb_body, 0)

        @pl.when(tid < 10)
        def _():
            for j in range(n_init):
                pltpu.sync_copy(
                    ones_v, cnt_sh.at[pl.ds(wbase + j * cch, cch)])

        def ones_body(r, carry):
            ones_v[r] = one16
            return carry
        lax.fori_loop(0, cch, ones_body, 0)

        plsc.subcore_barrier()

        ebase = wid * per_w

        def body(i, carry):
            off = ebase + i * cch
            pltpu.sync_copy(col_hbm.at[pl.ds(off, cch)], idx_v)
            pltpu.sync_copy(ones_v, cnt_sh.at[idx_v], add=True)
            return carry

        lax.fori_loop(0, steps, body, 0)

        plsc.subcore_barrier()

        @pl.when(tid < 10)
        def _():
            pltpu.sync_copy(cnt_sh.at[pl.ds(wbase, n_wr)],
                            cnt_hbm.at[core, pl.ds(wbase, n_wr)])

    return k(col)


def kernel(x, edge_index, edge_attr, u, batch, W1, b1, g1, be1, W2, b2, g2, be2):
    n, d = x.shape
    row = edge_index[0]
    col = edge_index[1]

    e = row.shape[0]
    part = (e * 2 // 5) // 6400 * 6400
    bounds = [0, part, 2 * part, e]

    attr_row = edge_attr.reshape(1, e)
    cnt16 = _sc_counts(col, n)
    H = _mm_bias_pack(x, W1[:d], b1, block=1000)
    pre = _node_pre(x, u, batch, W2, b2, block=1000)
    s_list = []
    for lo, hi in zip(bounds[:-1], bounds[1:]):
        Hrow = _sc_gather(H, row[lo:hi])
        Eout = _edge_elem(Hrow, attr_row[:, lo:hi], W1[d], g1, be1,
                          block=1600)
        s_list.append(_sc_scatter(Eout, col[lo:hi], n))
    return _node_mlp(pre, s_list, cnt16, W2, g2, be2, block=1000)


# speedup vs baseline: 1.0184x; 1.0184x over previous
import functools

import jax
import jax.numpy as jnp
from jax import lax
from jax.experimental import pallas as pl
from jax.experimental.pallas import tpu as pltpu
from jax.experimental.pallas import tpu_sc as plsc

_EPS = 1e-5


def _mm_bias_pack_body(x_ref, w_ref, b_ref, o_ref):
    h = (jnp.dot(x_ref[:, :], w_ref[:, :], preferred_element_type=jnp.float32)
         + b_ref[:, :])
    half = h.shape[1] // 2
    ai = lax.bitcast_convert_type(h[:, :half], jnp.int32)
    bi = lax.bitcast_convert_type(h[:, half:], jnp.int32)
    ta = (ai + 0x7FFF + ((ai >> 16) & 1)) >> 16
    tb = (bi + 0x7FFF + ((bi >> 16) & 1)) >> 16
    word = (tb << 16) | (ta & 0xFFFF)
    o_ref[:, :] = lax.bitcast_convert_type(word, jnp.float32)


def _mm_bias_pack(x, W, b, block):
    n, k = x.shape
    m = W.shape[1]
    return pl.pallas_call(
        _mm_bias_pack_body,
        grid=(n // block,),
        in_specs=[
            pl.BlockSpec((block, k), lambda i: (i, 0)),
            pl.BlockSpec((k, m), lambda i: (0, 0)),
            pl.BlockSpec((1, m), lambda i: (0, 0)),
        ],
        out_specs=pl.BlockSpec((block, m // 2), lambda i: (i, 0)),
        out_shape=jax.ShapeDtypeStruct((n, m // 2), jnp.float32),
    )(x, W, b.reshape(1, m))


def _ln(r, g, be):
    m = jnp.mean(r, axis=1, keepdims=True)
    d = r - m
    v = jnp.mean(d * d, axis=1, keepdims=True)
    return d * lax.rsqrt(v + _EPS) * g + be


def _edge_body(h_ref, a_ref, wl_ref, g_ref, be_ref, o_ref):
    w = lax.bitcast_convert_type(h_ref[:, :], jnp.int32)
    lo = lax.bitcast_convert_type(w << 16, jnp.float32)
    hi = lax.bitcast_convert_type(w & jnp.int32(-65536), jnp.float32)
    h = jnp.concatenate([lo, hi], axis=1)
    contrib = lax.dot_general(a_ref[0], wl_ref[:, :],
                              (((0,), (0,)), ((), ())),
                              preferred_element_type=jnp.float32)
    z = h + contrib
    r = jnp.maximum(z, 0.0)
    o_ref[:, :] = _ln(r, g_ref[:, :], be_ref[:, :])


def _edge_elem(Hrow, attr_row, w1l, g1, be1, block):
    e, dp = Hrow.shape
    d = dp * 2
    attr3 = attr_row.reshape(e // block, 1, block)
    return pl.pallas_call(
        _edge_body,
        grid=(e // block,),
        in_specs=[
            pl.BlockSpec((block, dp), lambda i: (i, 0)),
            pl.BlockSpec((1, 1, block), lambda i: (i, 0, 0)),
            pl.BlockSpec((1, d), lambda i: (0, 0)),
            pl.BlockSpec((1, d), lambda i: (0, 0)),
            pl.BlockSpec((1, d), lambda i: (0, 0)),
        ],
        out_specs=pl.BlockSpec((block, d), lambda i: (i, 0)),
        out_shape=jax.ShapeDtypeStruct((e, d), jnp.float32),
    )(Hrow, attr3, w1l.reshape(1, d), g1.reshape(1, d), be1.reshape(1, d))


def _node_pre_body(x_ref, bt_ref, u_ref, wx_ref, wu_ref, b_ref, o_ref):
    gsize = u_ref.shape[1]
    onehot = jnp.where(
        bt_ref[:, :] == lax.broadcasted_iota(jnp.int32, (1, gsize), 1), 1.0, 0.0)
    ub = jnp.sum(onehot * u_ref[:, :], axis=1, keepdims=True)
    o_ref[:, :] = (
        jnp.dot(x_ref[:, :], wx_ref[:, :], preferred_element_type=jnp.float32)
        + ub * wu_ref[:, :] + b_ref[:, :])


def _node_pre(x, u, batch, W2, b2, block):
    n, d = x.shape
    g = u.shape[0]
    wx = W2[:d]
    wu = W2[2 * d:2 * d + 1].reshape(1, d)
    return pl.pallas_call(
        _node_pre_body,
        grid=(n // block,),
        in_specs=[
            pl.BlockSpec((block, d), lambda i: (i, 0)),
            pl.BlockSpec((block, 1), lambda i: (i, 0)),
            pl.BlockSpec((1, g), lambda i: (0, 0)),
            pl.BlockSpec((d, d), lambda i: (0, 0)),
            pl.BlockSpec((1, d), lambda i: (0, 0)),
            pl.BlockSpec((1, d), lambda i: (0, 0)),
        ],
        out_specs=pl.BlockSpec((block, d), lambda i: (i, 0)),
        out_shape=jax.ShapeDtypeStruct((n, d), jnp.float32),
    )(x, batch.reshape(n, 1), u.reshape(1, g), wx, wu, b2.reshape(1, d))


def _node_stage(base, s_list, cnt16, wm, g2, be2, block, final):
    n, d = base.shape

    def body(*refs):
        nparts = len(refs) - 4 - (2 if final else 0)
        p_ref = refs[0]
        s_refs = refs[1:1 + nparts]
        rest = refs[1 + nparts:]
        c_ref, wm_ref = rest[0], rest[1]
        o_ref = rest[-1]
        cnt = c_ref[0, :, :1] + c_ref[1, :, :1]
        s = s_refs[0][:, :].astype(jnp.float32)
        for sr in s_refs[1:]:
            s = s + sr[:, :].astype(jnp.float32)
        mean = s / jnp.maximum(cnt, 1.0)
        z = p_ref[:, :] + jnp.dot(mean, wm_ref[:, :],
                                  preferred_element_type=jnp.float32)
        if final:
            r = jnp.maximum(z, 0.0)
            o_ref[:, :] = _ln(r, rest[2][:, :], rest[3][:, :])
        else:
            o_ref[:, :] = z

    extra = []
    extra_specs = []
    if final:
        extra = [g2.reshape(1, d), be2.reshape(1, d)]
        extra_specs = [pl.BlockSpec((1, d), lambda i: (0, 0)),
                       pl.BlockSpec((1, d), lambda i: (0, 0))]
    return pl.pallas_call(
        body,
        grid=(n // block,),
        in_specs=[
            pl.BlockSpec((block, d), lambda i: (i, 0)),
            *[pl.BlockSpec((block, d), lambda i: (i, 0)) for _ in s_list],
            pl.BlockSpec((2, block, 16), lambda i: (0, i, 0)),
            pl.BlockSpec((d, d), lambda i: (0, 0)),
            *extra_specs,
        ],
        out_specs=pl.BlockSpec((block, d), lambda i: (i, 0)),
        out_shape=jax.ShapeDtypeStruct((n, d), jnp.float32),
    )(base, *s_list, cnt16, wm, *extra)


_NW = 32
_GCHUNK = 200


def _sc_gather(H, row):
    e = row.shape[0]
    d = H.shape[1]
    per_w = e // _NW
    steps = per_w // _GCHUNK
    mesh = plsc.VectorSubcoreMesh(core_axis_name="c", subcore_axis_name="s")

    ch = _GCHUNK
    assert steps >= 4
    odd = steps % 2 == 1

    @functools.partial(
        pl.kernel, mesh=mesh,
        out_type=jax.ShapeDtypeStruct((e, d), jnp.float32),
        compiler_params=pltpu.CompilerParams(use_tc_tiling_on_sc=True),
        scratch_types=[
            pltpu.VMEM((ch,), jnp.int32),
            pltpu.VMEM((ch,), jnp.int32),
            pltpu.VMEM((ch, d), jnp.float32),
            pltpu.VMEM((ch, d), jnp.float32),
            pltpu.SemaphoreType.DMA,
            pltpu.SemaphoreType.DMA,
            pltpu.SemaphoreType.DMA,
            pltpu.SemaphoreType.DMA,
        ],
    )
    def k(h_hbm, row_hbm, out_hbm, ia, ib, ra, rb, sga, sgb, swa, swb):
        wid = lax.axis_index("s") * 2 + lax.axis_index("c")
        base = wid * per_w

        def off(kk):
            return pl.multiple_of(base + kk * ch, 8)

        def load_idx(kk, iv):
            pltpu.sync_copy(row_hbm.at[pl.ds(off(kk), ch)], iv)

        def start_g(iv, rv, sem):
            pltpu.async_copy(h_hbm.at[iv], rv, sem)

        def wait_g(iv, rv, sem):
            pltpu.make_async_copy(h_hbm.at[iv], rv, sem).wait()

        def start_w(kk, rv, sem):
            pltpu.async_copy(rv, out_hbm.at[pl.ds(off(kk), ch)], sem)

        def wait_w(kk, rv, sem):
            pltpu.make_async_copy(rv, out_hbm.at[pl.ds(off(kk), ch)], sem).wait()

        load_idx(0, ia)
        start_g(ia, ra, sga)
        load_idx(1, ib)
        start_g(ib, rb, sgb)

        def body(m, carry):
            k0 = 2 * m
            k1 = k0 + 1
            wait_g(ia, ra, sga)
            start_w(k0, ra, swa)
            wait_g(ib, rb, sgb)
            start_w(k1, rb, swb)
            load_idx(k0 + 2, ia)
            wait_w(k0, ra, swa)
            start_g(ia, ra, sga)
            load_idx(k1 + 2, ib)
            wait_w(k1, rb, swb)
            start_g(ib, rb, sgb)
            return carry

        if odd:
            lax.fori_loop(0, (steps - 3) // 2, body, 0)
            k2 = steps - 3
            wait_g(ia, ra, sga)
            start_w(k2, ra, swa)
            load_idx(steps - 1, ia)
            wait_w(k2, ra, swa)
            start_g(ia, ra, sga)
            wait_g(ib, rb, sgb)
            start_w(k2 + 1, rb, swb)
            wait_g(ia, ra, sga)
            start_w(k2 + 2, ra, swa)
            wait_w(k2 + 2, ra, swa)
            wait_w(k2 + 1, rb, swb)
        else:
            lax.fori_loop(0, (steps - 2) // 2, body, 0)
            wait_g(ia, ra, sga)
            start_w(steps - 2, ra, swa)
            wait_g(ib, rb, sgb)
            start_w(steps - 1, rb, swb)
            wait_w(steps - 2, ra, swa)
            wait_w(steps - 1, rb, swb)

    return k(H, row)


_SCHUNK = 200
_DCHUNK = 192


def _sc_scatter(Eout, col, n):
    e, d = Eout.shape
    half = d // 2
    per_tile = e // 16
    ch = next(c for c in range(_DCHUNK, 40, -8) if per_tile % c == 0)
    steps = per_tile // ch
    assert steps >= 4
    odd = steps % 2 == 1
    n_wr = n // 10
    q_init = n_wr // ch
    r_init = n_wr % ch
    assert r_init % 8 == 0
    mesh = plsc.VectorSubcoreMesh(core_axis_name="c", subcore_axis_name="s")

    @functools.partial(
        pl.kernel, mesh=mesh,
        out_type=jax.ShapeDtypeStruct((n, d), jnp.float32),
        compiler_params=pltpu.CompilerParams(use_tc_tiling_on_sc=True),
        scratch_types=[
            pltpu.VMEM((ch,), jnp.int32),
            pltpu.VMEM((ch,), jnp.int32),
            pltpu.VMEM((ch, half), jnp.float32),
            pltpu.VMEM((ch, half), jnp.float32),
            pltpu.VMEM_SHARED((n, half), jnp.float32),
            pltpu.SemaphoreType.DMA,
            pltpu.SemaphoreType.DMA,
            pltpu.SemaphoreType.DMA,
            pltpu.SemaphoreType.DMA,
        ],
    )
    def k(eo_hbm, col_hbm, s_hbm, ia, ib, da, db, acc_sh, sra, srb, saa, sab):
        tid = lax.axis_index("s")
        core = lax.axis_index("c")
        f0 = pl.multiple_of(core * half, half)
        wbase = pl.multiple_of(tid * n_wr, 8)

        zero16 = jnp.zeros((16,), jnp.float32)

        def zdat_body(r, carry):
            for c8 in range(half // 16):
                da[r, pl.ds(c8 * 16, 16)] = zero16
            return carry
        lax.fori_loop(0, ch, zdat_body, 0)

        @pl.when(tid < 10)
        def _():
            for j in range(q_init):
                pltpu.sync_copy(da, acc_sh.at[pl.ds(wbase + j * ch, ch)])
            if r_init:
                pltpu.sync_copy(
                    da.at[pl.ds(0, r_init)],
                    acc_sh.at[pl.ds(wbase + q_init * ch, r_init)])

        plsc.subcore_barrier()

        ebase = tid * per_tile

        def off(kk):
            return pl.multiple_of(ebase + kk * ch, 8)

        def fetch(kk, iv, dv, sem):
            pltpu.sync_copy(col_hbm.at[pl.ds(off(kk), ch)], iv)
            pltpu.async_copy(
                eo_hbm.at[pl.ds(off(kk), ch), pl.ds(f0, half)], dv, sem)

        def wait_r(kk, dv, sem):
            pltpu.make_async_copy(
                eo_hbm.at[pl.ds(off(kk), ch), pl.ds(f0, half)], dv,
                sem).wait()

        def start_a(iv, dv, sem):
            pltpu.async_copy(dv, acc_sh.at[iv], sem, add=True)

        def wait_a(iv, dv, sem):
            pltpu.make_async_copy(dv, acc_sh.at[iv], sem).wait()

        fetch(0, ia, da, sra)
        fetch(1, ib, db, srb)

        def body(m, carry):
            k0 = 2 * m
            k1 = k0 + 1
            wait_r(k0, da, sra)
            start_a(ia, da, saa)
            wait_r(k1, db, srb)
            start_a(ib, db, sab)
            wait_a(ia, da, saa)
            fetch(k0 + 2, ia, da, sra)
            wait_a(ib, db, sab)
            fetch(k1 + 2, ib, db, srb)
            return carry

        if odd:
            lax.fori_loop(0, (steps - 3) // 2, body, 0)
            k2 = steps - 3
            wait_r(k2, da, sra)
            start_a(ia, da, saa)
            wait_a(ia, da, saa)
            fetch(steps - 1, ia, da, sra)
            wait_r(k2 + 1, db, srb)
            start_a(ib, db, sab)
            wait_r(k2 + 2, da, sra)
            start_a(ia, da, saa)
            wait_a(ia, da, saa)
            wait_a(ib, db, sab)
        else:
            lax.fori_loop(0, (steps - 2) // 2, body, 0)
            wait_r(steps - 2, da, sra)
            start_a(ia, da, saa)
            wait_r(steps - 1, db, srb)
            start_a(ib, db, sab)
            wait_a(ia, da, saa)
            wait_a(ib, db, sab)

        plsc.subcore_barrier()

        @pl.when(tid < 10)
        def _():
            pltpu.sync_copy(acc_sh.at[pl.ds(wbase, n_wr)],
                            s_hbm.at[pl.ds(wbase, n_wr), pl.ds(f0, half)])

    return k(Eout, col)


def _sc_counts(col, n):
    e = col.shape[0]
    per_w = e // _NW
    cch = 1000
    steps = per_w // cch
    n_wr = n // 10
    mesh = plsc.VectorSubcoreMesh(core_axis_name="c", subcore_axis_name="s")

    @functools.partial(
        pl.kernel, mesh=mesh,
        out_type=jax.ShapeDtypeStruct((2, n, 16), jnp.float32),
        compiler_params=pltpu.CompilerParams(use_tc_tiling_on_sc=False),
        scratch_types=[
            pltpu.VMEM((cch,), jnp.int32),
            pltpu.VMEM((cch, 16), jnp.float32),
            pltpu.VMEM_SHARED((n, 16), jnp.float32),
        ],
    )
    def k(col_hbm, cnt_hbm, idx_v, ones_v, cnt_sh):
        tid = lax.axis_index("s")
        core = lax.axis_index("c")
        wid = tid * 2 + core
        wbase = pl.multiple_of(tid * n_wr, 8)

        zero16 = jnp.zeros((16,), jnp.float32)
        one16 = jnp.ones((16,), jnp.float32)
        n_init = n_wr // cch

        def zb_body(r, carry):
            ones_v[r] = zero16
            return carry
        lax.fori_loop(0, cch, zb_body, 0)

        @pl.when(tid < 10)
        def _():
            for j in range(n_init):
                pltpu.sync_copy(
                    ones_v, cnt_sh.at[pl.ds(wbase + j * cch, cch)])

        def ones_body(r, carry):
            ones_v[r] = one16
            return carry
        lax.fori_loop(0, cch, ones_body, 0)

        plsc.subcore_barrier()

        ebase = wid * per_w

        def body(i, carry):
            off = ebase + i * cch
            pltpu.sync_copy(col_hbm.at[pl.ds(off, cch)], idx_v)
            pltpu.sync_copy(ones_v, cnt_sh.at[idx_v], add=True)
            return carry

        lax.fori_loop(0, steps, body, 0)

        plsc.subcore_barrier()

        @pl.when(tid < 10)
        def _():
            pltpu.sync_copy(cnt_sh.at[pl.ds(wbase, n_wr)],
                            cnt_hbm.at[core, pl.ds(wbase, n_wr)])

    return k(col)


def kernel(x, edge_index, edge_attr, u, batch, W1, b1, g1, be1, W2, b2, g2, be2):
    n, d = x.shape
    row = edge_index[0]
    col = edge_index[1]

    e = row.shape[0]
    part = (e // 3) // 6400 * 6400
    bounds = [0, part, 2 * part, e]

    attr_row = edge_attr.reshape(1, e)
    cnt16 = _sc_counts(col, n)
    H = _mm_bias_pack(x, W1[:d], b1, block=1000)
    pre = _node_pre(x, u, batch, W2, b2, block=1000)
    s_list = []
    for lo, hi in zip(bounds[:-1], bounds[1:]):
        Hrow = _sc_gather(H, row[lo:hi])
        Eout = _edge_elem(Hrow, attr_row[:, lo:hi], W1[d], g1, be1,
                          block=1600)
        s_list.append(_sc_scatter(Eout, col[lo:hi], n))
    wm = W2[d:2 * d]
    zmid = _node_stage(pre, s_list[:-1], cnt16, wm, g2, be2, 1000, False)
    return _node_stage(zmid, s_list[-1:], cnt16, wm, g2, be2, 1000, True)
